# baseline (speedup 1.0000x reference)
import functools

import jax
import jax.numpy as jnp
from jax import lax
from jax.experimental import pallas as pl
from jax.experimental.pallas import tpu as pltpu
from jax.experimental.pallas import tpu_sc as plsc

_NBUF = 7
_CHUNK_ROWS = 16


def _broadcast_copy_sc(weights, seq_len, bsz, embed_dim):
    info = plsc.get_sparse_core_info()
    num_workers = info.num_cores * info.num_subcores
    rows_per_w = seq_len // num_workers
    n_chunks = rows_per_w // _CHUNK_ROWS

    mesh = plsc.VectorSubcoreMesh(core_axis_name="c", subcore_axis_name="s")

    @functools.partial(
        pl.kernel,
        mesh=mesh,
        out_type=jax.ShapeDtypeStruct((seq_len, bsz, embed_dim), jnp.float32),
        scratch_types=[
            pltpu.VMEM((_NBUF, _CHUNK_ROWS, embed_dim), jnp.float32),
            pltpu.SemaphoreType.DMA,
            pltpu.SemaphoreType.DMA,
        ],
    )
    def k(w_hbm, out_hbm, buf, in_sem, out_sem):
        wid = lax.axis_index("s") * info.num_cores + lax.axis_index("c")
        base = wid * rows_per_w

        def load(c):
            cp = pltpu.make_async_copy(
                w_hbm.at[pl.ds(base + c * _CHUNK_ROWS, _CHUNK_ROWS)],
                buf.at[c % _NBUF],
                in_sem,
            )
            cp.start()
            return cp

        def stores(c):
            cps = []
            for b in range(bsz):
                cp = pltpu.make_async_copy(
                    buf.at[c % _NBUF],
                    out_hbm.at[pl.ds(base + c * _CHUNK_ROWS, _CHUNK_ROWS), b],
                    out_sem,
                )
                cp.start()
                cps.append(cp)
            return cps

        pending_stores = [None] * n_chunks
        pending_loads = [None] * n_chunks
        for c in range(n_chunks):
            if c >= _NBUF:
                for cp in pending_stores[c - _NBUF]:
                    cp.wait()
            pending_loads[c] = load(c)
            if c >= 1:
                pending_loads[c - 1].wait()
                pending_stores[c - 1] = stores(c - 1)
        pending_loads[n_chunks - 1].wait()
        pending_stores[n_chunks - 1] = stores(n_chunks - 1)
        for c in range(max(0, n_chunks - _NBUF), n_chunks):
            for cp in pending_stores[c]:
                cp.wait()

    return k(weights)


def kernel(input, weights):
    seq_len, bsz = input.shape
    init_size, embed_dim = weights.shape
    return _broadcast_copy_sc(weights, seq_len, bsz, embed_dim)


# speedup vs baseline: 1.1102x; 1.1102x over previous
import functools

import jax
import jax.numpy as jnp
from jax import lax
from jax.experimental import pallas as pl
from jax.experimental.pallas import tpu as pltpu
from jax.experimental.pallas import tpu_sc as plsc

_NBUF = 3
_CHUNK_ROWS = 40


def _broadcast_copy_sc(weights, seq_len, bsz, embed_dim):
    info = plsc.get_sparse_core_info()
    num_workers = info.num_cores * info.num_subcores
    rows_per_w = seq_len // num_workers
    sizes = [_CHUNK_ROWS] * (rows_per_w // _CHUNK_ROWS)
    if rows_per_w % _CHUNK_ROWS:
        sizes.append(rows_per_w % _CHUNK_ROWS)
    offs = [sum(sizes[:i]) for i in range(len(sizes))]
    n_chunks = len(sizes)

    mesh = plsc.VectorSubcoreMesh(core_axis_name="c", subcore_axis_name="s")

    @functools.partial(
        pl.kernel,
        mesh=mesh,
        out_type=jax.ShapeDtypeStruct((seq_len, bsz, embed_dim), jnp.float32),
        scratch_types=[
            pltpu.VMEM((_NBUF, _CHUNK_ROWS, embed_dim), jnp.float32),
            pltpu.SemaphoreType.DMA,
            pltpu.SemaphoreType.DMA,
        ],
    )
    def k(w_hbm, out_hbm, buf, in_sem, out_sem):
        wid = lax.axis_index("s") * info.num_cores + lax.axis_index("c")
        base = wid * rows_per_w

        def load(c):
            cp = pltpu.make_async_copy(
                w_hbm.at[pl.ds(base + offs[c], sizes[c])],
                buf.at[c % _NBUF, pl.ds(0, sizes[c])],
                in_sem,
            )
            cp.start()
            return cp

        def stores(c):
            cps = []
            for b in range(bsz):
                cp = pltpu.make_async_copy(
                    buf.at[c % _NBUF, pl.ds(0, sizes[c])],
                    out_hbm.at[pl.ds(base + offs[c], sizes[c]), b],
                    out_sem,
                )
                cp.start()
                cps.append(cp)
            return cps

        pending_stores = [None] * n_chunks
        pending_loads = [None] * n_chunks
        for c in range(n_chunks):
            if c >= _NBUF:
                for cp in pending_stores[c - _NBUF]:
                    cp.wait()
            pending_loads[c] = load(c)
            if c >= 1:
                pending_loads[c - 1].wait()
                pending_stores[c - 1] = stores(c - 1)
        pending_loads[n_chunks - 1].wait()
        pending_stores[n_chunks - 1] = stores(n_chunks - 1)
        for c in range(max(0, n_chunks - _NBUF), n_chunks):
            for cp in pending_stores[c]:
                cp.wait()

    return k(weights)


def kernel(input, weights):
    seq_len, bsz = input.shape
    init_size, embed_dim = weights.shape
    return _broadcast_copy_sc(weights, seq_len, bsz, embed_dim)
